# baseline (speedup 1.0000x reference)
import functools

import jax
import jax.numpy as jnp
from jax import lax
from jax.experimental import pallas as pl
from jax.experimental.pallas import tpu as pltpu
from jax.experimental.pallas import tpu_sc as plsc

N_FACTORS = 32
BATCH = 16384
LANES = 16
CHUNK = 128

_info = plsc.get_sparse_core_info()
_NC, _NS = _info.num_cores, _info.num_subcores
_NW = _NC * _NS
_BPW = BATCH // _NW
_NCHUNK = _BPW // CHUNK


def _sc_body(user_ref, item_ref, uf_ref, if_ref, out_ref,
             idx_u, idx_i, u_rows, v_rows, p_buf, out_v, sem):
    wid = lax.axis_index("s") * _NC + lax.axis_index("c")
    base_chunk = wid * _NCHUNK

    pltpu.sync_copy(user_ref.at[pl.ds(base_chunk, _NCHUNK)], idx_u)
    pltpu.sync_copy(item_ref.at[pl.ds(base_chunk, _NCHUNK)], idx_i)

    copies = []
    for j in range(_NCHUNK):
        copies.append(pltpu.async_copy(
            uf_ref.at[idx_u.at[j]], u_rows.at[pl.ds(j * CHUNK, CHUNK)], sem))
        copies.append(pltpu.async_copy(
            if_ref.at[idx_i.at[j]], v_rows.at[pl.ds(j * CHUNK, CHUNK)], sem))
    for c in copies:
        c.wait()

    lane_ids = lax.iota(jnp.int32, LANES) * LANES
    half = N_FACTORS // 2

    def group(g, carry):
        r0 = g * LANES
        for i in range(LANES):
            r = r0 + i
            u0 = u_rows[r, pl.ds(0, half)]
            u1 = u_rows[r, pl.ds(half, half)]
            v0 = v_rows[r, pl.ds(0, half)]
            v1 = v_rows[r, pl.ds(half, half)]
            p_buf[pl.ds(i * LANES, LANES)] = u0 * v0 + u1 * v1
        acc = jnp.zeros((LANES,), jnp.float32)
        for l in range(LANES):
            acc = acc + plsc.load_gather(p_buf, [lane_ids + l])
        out_v[pl.ds(r0, LANES)] = acc
        return carry

    lax.fori_loop(0, _BPW // LANES, group, 0, unroll=False)

    pltpu.sync_copy(out_v, out_ref.at[pl.ds(wid * _BPW, _BPW)])


@jax.jit
def _sc_call(user2d, item2d, user_factors, item_factors):
    mesh = plsc.VectorSubcoreMesh(core_axis_name="c", subcore_axis_name="s")
    return pl.kernel(
        _sc_body,
        mesh=mesh,
        compiler_params=pltpu.CompilerParams(
            needs_layout_passes=False, use_tc_tiling_on_sc=False),
        out_type=jax.ShapeDtypeStruct((BATCH,), jnp.float32),
        scratch_types=[
            pltpu.VMEM((_NCHUNK, CHUNK), jnp.int32),
            pltpu.VMEM((_NCHUNK, CHUNK), jnp.int32),
            pltpu.VMEM((_BPW, N_FACTORS), jnp.float32),
            pltpu.VMEM((_BPW, N_FACTORS), jnp.float32),
            pltpu.VMEM((LANES * LANES,), jnp.float32),
            pltpu.VMEM((_BPW,), jnp.float32),
            pltpu.SemaphoreType.DMA,
        ],
    )(user2d, item2d, user_factors, item_factors)


def kernel(user, item, user_factors, item_factors):
    user2d = user.astype(jnp.int32).reshape(BATCH // CHUNK, CHUNK)
    item2d = item.astype(jnp.int32).reshape(BATCH // CHUNK, CHUNK)
    return _sc_call(user2d, item2d, user_factors, item_factors)
